# Initial kernel scaffold; baseline (speedup 1.0000x reference)
import jax
import jax.numpy as jnp
from jax.experimental import pallas as pl


def kernel(x0, mask, table, ln_gamma, ln_beta):
    raise NotImplementedError("write your pallas kernel here")


# speedup vs baseline: 1.9057x; 1.9057x over previous
import functools

import jax
import jax.numpy as jnp
from jax import lax
from jax.experimental import pallas as pl
from jax.experimental.pallas import tpu as pltpu
from jax.experimental.pallas import tpu_sc as plsc

_IN_DIM = 1000000
_D = 64
_EPS = 1e-5
_L = 16
_W = 128


def _rsqrt(v):
    bits = lax.bitcast_convert_type(v, jnp.int32)
    y = lax.bitcast_convert_type(jnp.int32(0x5F3759DF) - (bits >> 1),
                                 jnp.float32)
    vh = v * 0.5
    y = y * (1.5 - vh * y * y)
    y = y * (1.5 - vh * y * y)
    return y


def kernel(x0, mask, table, ln_gamma, ln_beta):
    B, F = x0.shape
    N = B * F
    x0f = x0.reshape(1, N).astype(jnp.int32)
    mf = mask.reshape(1, N).astype(jnp.int32)
    gb = jnp.stack([ln_gamma, ln_beta]).astype(jnp.float32)

    mesh = plsc.VectorSubcoreMesh(core_axis_name="c", subcore_axis_name="s")

    @functools.partial(
        pl.kernel,
        out_type=jax.ShapeDtypeStruct((N, _D), jnp.float32),
        mesh=mesh,
        scratch_types=[pltpu.VMEM((_W,), jnp.int32),
                       pltpu.VMEM((2, _D), jnp.float32)],
        compiler_params=pltpu.CompilerParams(needs_layout_passes=False,
                                             use_tc_tiling_on_sc=False),
    )
    def run(x0_hbm, m_hbm, tab_hbm, gb_hbm, out_hbm, idx_v, gb_v):
        pltpu.sync_copy(gb_hbm, gb_v)

        def body(x0_vm, m_vm, o_vm):
            @pl.loop(0, _W, step=_L)
            def _(i):
                xv = x0_vm[0, pl.ds(i, _L)]
                mv = m_vm[0, pl.ds(i, _L)]
                idx_v[pl.ds(i, _L)] = jnp.where(mv != 0, _IN_DIM, xv)

            pltpu.sync_copy(tab_hbm.at[idx_v], o_vm)

            @pl.loop(0, _W)
            def _(r):
                v0 = o_vm[r, pl.ds(0, _L)]
                v1 = o_vm[r, pl.ds(_L, _L)]
                v2 = o_vm[r, pl.ds(2 * _L, _L)]
                v3 = o_vm[r, pl.ds(3 * _L, _L)]
                s = (v0 + v1) + (v2 + v3)
                sq = (v0 * v0 + v1 * v1) + (v2 * v2 + v3 * v3)
                st = jnp.sum(s)
                sqt = jnp.sum(sq)
                mean = st * (1.0 / _D)
                var = sqt * (1.0 / _D) - mean * mean + _EPS
                inv = _rsqrt(jnp.full((_L,), var, jnp.float32))
                mv_ = jnp.full((_L,), mean, jnp.float32)
                for j, vj in enumerate((v0, v1, v2, v3)):
                    g = gb_v[0, pl.ds(j * _L, _L)]
                    b = gb_v[1, pl.ds(j * _L, _L)]
                    o_vm[r, pl.ds(j * _L, _L)] = (vj - mv_) * inv * g + b

        pltpu.emit_pipeline(
            body,
            grid=(N // _W,),
            in_specs=[pl.BlockSpec((1, _W), lambda i: (0, i)),
                      pl.BlockSpec((1, _W), lambda i: (0, i))],
            out_specs=[pl.BlockSpec((_W, _D), lambda i: (i, 0))],
            core_axis_name=("c", "s"),
            dimension_semantics=(pltpu.PARALLEL,),
        )(x0_hbm, m_hbm, out_hbm)

    out = run(x0f, mf, table, gb)
    return out.reshape(B, F, _D)
